# Initial kernel scaffold; baseline (speedup 1.0000x reference)
import jax
import jax.numpy as jnp
from jax.experimental import pallas as pl


def kernel(nodes, features, neigh_idx, W1, W2, W3):
    raise NotImplementedError("write your pallas kernel here")


# speedup vs baseline: 12.1619x; 12.1619x over previous
import functools

import jax
import jax.numpy as jnp
from jax import lax
from jax.experimental import pallas as pl
from jax.experimental.pallas import tpu as pltpu
from jax.experimental.pallas import tpu_sc as plsc

N = 10000
NP = 10240
D = 256
DH = 128
B = 4096
C = 40
K1 = 25
K2 = 10
KPAD = 64
KPAD2 = 128

_NC, _NS, _L = 2, 16, 16
_NW = _NC * _NS
_S = 8
_GH = DH // _L


def _tc_matmul(x, w):
    R = x.shape[0]
    blk = 400 if R % 400 == 0 else 512

    def body(x_ref, w_ref, o_ref):
        o_ref[...] = jnp.dot(x_ref[...], w_ref[...],
                             preferred_element_type=jnp.float32)

    return pl.pallas_call(
        body,
        grid=(R // blk,),
        in_specs=[
            pl.BlockSpec((blk, D), lambda i: (i, 0)),
            pl.BlockSpec((D, D), lambda i: (0, 0)),
        ],
        out_specs=pl.BlockSpec((blk, D), lambda i: (i, 0)),
        out_shape=jax.ShapeDtypeStruct((R, D), jnp.float32),
    )(x, w)


def _tc_head(x, w2, w3):
    blk = 512

    def body(x_ref, w2_ref, w3_ref, o_ref):
        h = jnp.maximum(
            jnp.dot(x_ref[...], w2_ref[...], preferred_element_type=jnp.float32),
            0.0)
        o_ref[...] = jnp.dot(h, w3_ref[...], preferred_element_type=jnp.float32)

    return pl.pallas_call(
        body,
        grid=(B // blk,),
        in_specs=[
            pl.BlockSpec((blk, D), lambda i: (i, 0)),
            pl.BlockSpec((D, D), lambda i: (0, 0)),
            pl.BlockSpec((D, C), lambda i: (0, 0)),
        ],
        out_specs=pl.BlockSpec((blk, C), lambda i: (i, 0)),
        out_shape=jax.ShapeDtypeStruct((B, C), jnp.float32),
    )(x, w2, w3)


def _pipelined_gather_sum(table_hbm, idx_v, idx_off, out_v, bufs, sems,
                          nrows, krows, relu):
    kd = 2 * krows

    def fire(n, slot):
        pltpu.async_copy(table_hbm.at[idx_v.at[idx_off + n, pl.ds(0, kd)]],
                         bufs[slot], sems[slot])

    def wait(n, slot):
        pltpu.make_async_copy(
            table_hbm.at[idx_v.at[idx_off + n, pl.ds(0, kd)]],
            bufs[slot], sems[slot]).wait()

    def process(n, slot):
        buf = bufs[slot]

        def rbody(r, accs):
            lo = tuple(accs[c] + buf[2 * r, pl.ds(c * _L, _L)]
                       for c in range(_GH))
            hi = tuple(accs[_GH + c] + buf[2 * r + 1, pl.ds(c * _L, _L)]
                       for c in range(_GH))
            return lo + hi

        accs = (tuple(buf[0, pl.ds(c * _L, _L)] for c in range(_GH))
                + tuple(buf[1, pl.ds(c * _L, _L)] for c in range(_GH)))
        accs = lax.fori_loop(1, krows, rbody, accs, unroll=4)
        for c in range(2 * _GH):
            a = accs[c]
            if relu:
                a = jnp.maximum(a, 0.0)
            out_v[n, pl.ds(c * _L, _L)] = a

    for j in range(6):
        fire(jnp.int32(j), j)
    for j in range(4):
        wait(jnp.int32(j), j)

    def group(g, _):
        n0 = g * _S
        for b in range(_S):
            n = n0 + b
            fire(n + 6, (b + 6) % _S)
            wait(n + 4, (b + 4) % _S)
            process(n, b)
        return 0

    lax.fori_loop(0, (nrows - _S) // _S, group, 0)

    for b in range(_S):
        n = jnp.int32(nrows - _S + b)
        if b < 2:
            fire(n + 6, (b + 6) % _S)
        elif b < 6:
            fire(jnp.int32(0), (b + 6) % _S)
        if b < 4:
            wait(n + 4, (b + 4) % _S)
        else:
            wait(jnp.int32(0), (b + 4) % _S)
        process(n, b)


def _sc_gather1(fproj2, nidx):
    NPW = NP // _NW
    CH = NPW // 4
    mesh = plsc.VectorSubcoreMesh(core_axis_name="c", subcore_axis_name="s")

    @functools.partial(
        pl.kernel, mesh=mesh,
        out_type=jax.ShapeDtypeStruct((NP, D), jnp.float32),
        scratch_types=[
            pltpu.VMEM((NPW, KPAD), jnp.int32),
        ] + [pltpu.VMEM((2 * K1, DH), jnp.float32)] * _S + [
            pltpu.VMEM((CH, D), jnp.float32),
        ] + [pltpu.SemaphoreType.DMA] * _S,
    )
    def k(fproj_hbm, nidx_hbm, out_hbm, idx_v, *rest):
        bufs, out_v, sems = rest[:_S], rest[_S], rest[_S + 1:]
        wid = lax.axis_index("s") * _NC + lax.axis_index("c")
        base = wid * NPW
        pltpu.sync_copy(nidx_hbm.at[pl.ds(base, NPW)], idx_v)
        pltpu.sync_copy(nidx_hbm.at[pl.ds(base, NPW)], idx_v)

        def chunk_body(ch, _):
            off = ch * CH
            _pipelined_gather_sum(fproj_hbm, idx_v, off, out_v, bufs, sems,
                                  CH, K1, relu=True)
            pltpu.sync_copy(out_v, out_hbm.at[pl.ds(base + off, CH)])
            return 0

        lax.fori_loop(0, 4, chunk_body, 0)

    return k(fproj2, nidx)


def _sc_gather2(h1t2, nidx2, nodes):
    BPW = B // _NW
    mesh = plsc.VectorSubcoreMesh(core_axis_name="c", subcore_axis_name="s")

    @functools.partial(
        pl.kernel, mesh=mesh,
        out_type=jax.ShapeDtypeStruct((B, D), jnp.float32),
        scratch_types=[
            pltpu.VMEM((BPW,), jnp.int32),
            pltpu.VMEM((BPW, KPAD2), jnp.int32),
        ] + [pltpu.VMEM((2 * K2, DH), jnp.float32)] * _S + [
            pltpu.VMEM((BPW, D), jnp.float32),
        ] + [pltpu.SemaphoreType.DMA] * (_S + 1),
    )
    def k(h1t_hbm, nidx_hbm, nodes_hbm, out_hbm, nodes_v, nbrs_v, *rest):
        bufs, out_v, sn, sems = (rest[:_S], rest[_S], rest[_S + 1],
                                 rest[_S + 2:])
        wid = lax.axis_index("s") * _NC + lax.axis_index("c")
        base = wid * BPW
        pltpu.sync_copy(nodes_hbm.at[pl.ds(base, BPW)], nodes_v)
        pltpu.sync_copy(nodes_hbm.at[pl.ds(base, BPW)], nodes_v)
        pltpu.async_copy(nidx_hbm.at[nodes_v], nbrs_v, sn).wait()
        pltpu.async_copy(nidx_hbm.at[nodes_v], nbrs_v, sn).wait()
        _pipelined_gather_sum(h1t_hbm, nbrs_v, 0, out_v, bufs, sems,
                              BPW, K2, relu=False)
        pltpu.sync_copy(out_v, out_hbm.at[pl.ds(base, BPW)])

    return k(h1t2, nidx2, nodes)


def _double_idx(idx, width):
    r, k = idx.shape
    d = jnp.stack([2 * idx, 2 * idx + 1], axis=-1).reshape(r, 2 * k)
    return jnp.pad(d, ((0, 0), (0, width - 2 * k)))


def kernel(nodes, features, neigh_idx, W1, W2, W3):
    nodes = nodes.astype(jnp.int32)
    neigh_idx = neigh_idx.astype(jnp.int32)
    nidx = jnp.pad(_double_idx(neigh_idx[:, :K1], KPAD), ((0, NP - N), (0, 0)))
    nidx2 = _double_idx(neigh_idx[:, :K2], KPAD2)
    w1t = (W1 * (1.0 / K1)).T
    w2t = (W2 * (1.0 / K2)).T
    w3t = W3.T
    fproj = _tc_matmul(features, w1t)
    h1t = _sc_gather1(fproj.reshape(2 * N, DH), nidx)
    sum2 = _sc_gather2(h1t.reshape(2 * NP, DH), nidx2, nodes)
    return _tc_head(sum2, w2t, w3t)


# speedup vs baseline: 14.8356x; 1.2198x over previous
import functools

import jax
import jax.numpy as jnp
from jax import lax
from jax.experimental import pallas as pl
from jax.experimental.pallas import tpu as pltpu
from jax.experimental.pallas import tpu_sc as plsc

N = 10000
NP = 10240
D = 256
DH = 128
B = 4096
C = 40
K1 = 25
K2 = 10
KPAD = 64
KPAD2 = 128

_NC, _NS, _L = 2, 16, 16
_NW = _NC * _NS
_S = 8
_GH = DH // _L


def _tc_matmul(x, w):
    R = x.shape[0]
    blk = 400 if R % 400 == 0 else 512

    def body(x_ref, w_ref, o_ref):
        o_ref[...] = jnp.dot(x_ref[...], w_ref[...],
                             preferred_element_type=jnp.float32)

    return pl.pallas_call(
        body,
        grid=(R // blk,),
        in_specs=[
            pl.BlockSpec((blk, D), lambda i: (i, 0)),
            pl.BlockSpec((D, D), lambda i: (0, 0)),
        ],
        out_specs=pl.BlockSpec((blk, D), lambda i: (i, 0)),
        out_shape=jax.ShapeDtypeStruct((R, D), jnp.float32),
    )(x, w)


def _tc_head(x, w2, w3):
    blk = 512

    def body(x_ref, w2_ref, w3_ref, o_ref):
        h = jnp.maximum(
            jnp.dot(x_ref[...], w2_ref[...], preferred_element_type=jnp.float32),
            0.0)
        o_ref[...] = jnp.dot(h, w3_ref[...], preferred_element_type=jnp.float32)

    return pl.pallas_call(
        body,
        grid=(B // blk,),
        in_specs=[
            pl.BlockSpec((blk, D), lambda i: (i, 0)),
            pl.BlockSpec((D, D), lambda i: (0, 0)),
            pl.BlockSpec((D, C), lambda i: (0, 0)),
        ],
        out_specs=pl.BlockSpec((blk, C), lambda i: (i, 0)),
        out_shape=jax.ShapeDtypeStruct((B, C), jnp.float32),
    )(x, w2, w3)


def _pipelined_gather_sum(table_hbm, idx_v, idx_off, out_v, bufs, sems,
                          nrows, krows, relu):
    kd = 2 * krows

    def fire(n, slot):
        pltpu.async_copy(table_hbm.at[idx_v.at[idx_off + n, pl.ds(0, kd)]],
                         bufs[slot], sems[slot])

    def wait(n, slot):
        pltpu.make_async_copy(
            table_hbm.at[idx_v.at[idx_off + n, pl.ds(0, kd)]],
            bufs[slot], sems[slot]).wait()

    def process(n, slot):
        buf = bufs[slot]

        def rbody(r, accs):
            lo = tuple(accs[c] + buf[2 * r, pl.ds(c * _L, _L)]
                       for c in range(_GH))
            hi = tuple(accs[_GH + c] + buf[2 * r + 1, pl.ds(c * _L, _L)]
                       for c in range(_GH))
            return lo + hi

        accs = (tuple(buf[0, pl.ds(c * _L, _L)] for c in range(_GH))
                + tuple(buf[1, pl.ds(c * _L, _L)] for c in range(_GH)))
        accs = lax.fori_loop(1, krows, rbody, accs, unroll=4)
        for c in range(2 * _GH):
            a = accs[c]
            if relu:
                a = jnp.maximum(a, 0.0)
            out_v[n, pl.ds(c * _L, _L)] = a

    for j in range(6):
        fire(jnp.int32(j), j)
    for j in range(4):
        wait(jnp.int32(j), j)

    def group(g, _):
        n0 = g * _S
        for b in range(_S):
            n = n0 + b
            fire(n + 6, (b + 6) % _S)
            wait(n + 4, (b + 4) % _S)
            process(n, b)
        return 0

    lax.fori_loop(0, (nrows - _S) // _S, group, 0)

    for b in range(_S):
        n = jnp.int32(nrows - _S + b)
        if b < 2:
            fire(n + 6, (b + 6) % _S)
        elif b < 6:
            fire(jnp.int32(0), (b + 6) % _S)
        if b < 4:
            wait(n + 4, (b + 4) % _S)
        else:
            wait(jnp.int32(0), (b + 4) % _S)
        process(n, b)


_NF = 560
_NS_ = 80
_CH = 80
_FAST_CORE = 0


def _sc_gather1(fproj2, nidx):
    mesh = plsc.VectorSubcoreMesh(core_axis_name="c", subcore_axis_name="s")

    @functools.partial(
        pl.kernel, mesh=mesh,
        out_type=jax.ShapeDtypeStruct((NP, D), jnp.float32),
        scratch_types=[
            pltpu.VMEM((_CH, KPAD), jnp.int32),
        ] + [pltpu.VMEM((2 * K1, DH), jnp.float32)] * _S + [
            pltpu.VMEM((_CH, D), jnp.float32),
        ] + [pltpu.SemaphoreType.DMA] * _S,
    )
    def k(fproj_hbm, nidx_hbm, out_hbm, idx_v, *rest):
        bufs, out_v, sems = rest[:_S], rest[_S], rest[_S + 1:]
        cidx = lax.axis_index("c")
        sidx = lax.axis_index("s")
        on_fast = cidx == _FAST_CORE
        base = jnp.where(on_fast, sidx * _NF, _NS * _NF + sidx * _NS_)
        nchunks = jnp.where(on_fast, _NF // _CH, _NS_ // _CH)

        def chunk_body(ch, _):
            off = base + ch * _CH
            pltpu.sync_copy(nidx_hbm.at[pl.ds(off, _CH)], idx_v)
            pltpu.sync_copy(nidx_hbm.at[pl.ds(off, _CH)], idx_v)
            _pipelined_gather_sum(fproj_hbm, idx_v, 0, out_v, bufs, sems,
                                  _CH, K1, relu=True)
            pltpu.sync_copy(out_v, out_hbm.at[pl.ds(off, _CH)])
            return 0

        lax.fori_loop(0, nchunks, chunk_body, 0)

    return k(fproj2, nidx)


def _sc_gather2(h1t2, nidx2, nodes):
    BPW = B // _NW
    mesh = plsc.VectorSubcoreMesh(core_axis_name="c", subcore_axis_name="s")

    @functools.partial(
        pl.kernel, mesh=mesh,
        out_type=jax.ShapeDtypeStruct((B, D), jnp.float32),
        scratch_types=[
            pltpu.VMEM((BPW,), jnp.int32),
            pltpu.VMEM((BPW, KPAD2), jnp.int32),
        ] + [pltpu.VMEM((2 * K2, DH), jnp.float32)] * _S + [
            pltpu.VMEM((BPW, D), jnp.float32),
        ] + [pltpu.SemaphoreType.DMA] * (_S + 1),
    )
    def k(h1t_hbm, nidx_hbm, nodes_hbm, out_hbm, nodes_v, nbrs_v, *rest):
        bufs, out_v, sn, sems = (rest[:_S], rest[_S], rest[_S + 1],
                                 rest[_S + 2:])
        wid = lax.axis_index("s") * _NC + lax.axis_index("c")
        base = wid * BPW
        pltpu.sync_copy(nodes_hbm.at[pl.ds(base, BPW)], nodes_v)
        pltpu.sync_copy(nodes_hbm.at[pl.ds(base, BPW)], nodes_v)
        pltpu.async_copy(nidx_hbm.at[nodes_v], nbrs_v, sn).wait()
        pltpu.async_copy(nidx_hbm.at[nodes_v], nbrs_v, sn).wait()
        _pipelined_gather_sum(h1t_hbm, nbrs_v, 0, out_v, bufs, sems,
                              BPW, K2, relu=False)
        pltpu.sync_copy(out_v, out_hbm.at[pl.ds(base, BPW)])

    return k(h1t2, nidx2, nodes)


def _double_idx(idx, width):
    r, k = idx.shape
    d = jnp.stack([2 * idx, 2 * idx + 1], axis=-1).reshape(r, 2 * k)
    return jnp.pad(d, ((0, 0), (0, width - 2 * k)))


def kernel(nodes, features, neigh_idx, W1, W2, W3):
    nodes = nodes.astype(jnp.int32)
    neigh_idx = neigh_idx.astype(jnp.int32)
    nidx = jnp.pad(_double_idx(neigh_idx[:, :K1], KPAD), ((0, NP - N), (0, 0)))
    nidx2 = _double_idx(neigh_idx[:, :K2], KPAD2)
    w1t = (W1 * (1.0 / K1)).T
    w2t = (W2 * (1.0 / K2)).T
    w3t = W3.T
    fproj = _tc_matmul(features, w1t)
    h1t = _sc_gather1(fproj.reshape(2 * N, DH), nidx)
    sum2 = _sc_gather2(h1t.reshape(2 * NP, DH), nidx2, nodes)
    return _tc_head(sum2, w2t, w3t)
